# baseline (speedup 1.0000x reference)
import functools

import jax
import jax.numpy as jnp
from jax.experimental import pallas as pl
from jax.experimental.pallas import tpu as pltpu

_BN = 2048
_NSLOT = 4
_AHEAD = 3


def _in_copy(mem_hbm, mbuf, sems, jj, slot, rows):
    return pltpu.make_async_copy(
        mem_hbm.at[pl.ds(jj * _BN, rows), :],
        mbuf.at[slot, pl.ds(0, rows), :],
        sems.at[slot],
    )


def _body(feat_ref, tfeat_ref, params_ref, mem_hbm,
          out_t_ref, out_f_ref, sim_ref, mbuf, sems, *, N):
    j = pl.program_id(0)
    nsteps = pl.num_programs(0)
    last = nsteps - 1
    tail = N - last * _BN
    slot = jax.lax.rem(j, _NSLOT)

    @pl.when(j == 0)
    def _():
        for jj in range(_AHEAD):
            _in_copy(mem_hbm, mbuf, sems, jj, jj % _NSLOT, _BN).start()

    nxt = j + _AHEAD
    nslot = jax.lax.rem(nxt, _NSLOT)

    @pl.when(nxt < last)
    def _():
        _in_copy(mem_hbm, mbuf, sems, nxt, nslot, _BN).start()

    @pl.when(nxt == last)
    def _():
        _in_copy(mem_hbm, mbuf, sems, nxt, nslot, tail).start()

    @pl.when(j < last)
    def _():
        _in_copy(mem_hbm, mbuf, sems, j, slot, _BN).wait()

    @pl.when(j == last)
    def _():
        _in_copy(mem_hbm, mbuf, sems, j, slot, tail).wait()

    t = params_ref[0, 0]
    inv_t = 1.0 / t
    f = feat_ref[...]
    tf = tfeat_ref[...]
    m = mbuf[slot]
    dims = (((1,), (1,)), ((), ()))
    out_f_ref[...] = jax.lax.dot_general(
        m, f, dims, preferred_element_type=jnp.float32) * inv_t
    out_t_ref[...] = jax.lax.dot_general(
        m, tf, dims, preferred_element_type=jnp.float32) * (inv_t * inv_t)

    @pl.when(j == 0)
    def _():
        sim_ref[...] = jnp.sum(f * tf, axis=-1, keepdims=True)


def kernel(image_features, transformed_image_features, indices, memory, params):
    del indices
    B, D = image_features.shape
    N = memory.shape[0]
    nb = pl.cdiv(N, _BN)
    p2d = params.reshape(1, 2)
    out_t, out_f, sim = pl.pallas_call(
        functools.partial(_body, N=N),
        grid=(nb,),
        in_specs=[
            pl.BlockSpec((B, D), lambda j: (0, 0)),
            pl.BlockSpec((B, D), lambda j: (0, 0)),
            pl.BlockSpec((1, 2), lambda j: (0, 0)),
            pl.BlockSpec(memory_space=pl.ANY),
        ],
        out_specs=[
            pl.BlockSpec((_BN, B), lambda j: (j, 0)),
            pl.BlockSpec((_BN, B), lambda j: (j, 0)),
            pl.BlockSpec((B, 1), lambda j: (0, 0)),
        ],
        out_shape=[
            jax.ShapeDtypeStruct((N, B), jnp.float32),
            jax.ShapeDtypeStruct((N, B), jnp.float32),
            jax.ShapeDtypeStruct((B, 1), jnp.float32),
        ],
        scratch_shapes=[
            pltpu.VMEM((_NSLOT, _BN, D), jnp.float32),
            pltpu.SemaphoreType.DMA((_NSLOT,)),
        ],
        compiler_params=pltpu.CompilerParams(
            dimension_semantics=("arbitrary",),
        ),
    )(image_features, transformed_image_features, p2d, memory)
    return (out_t.T, out_f.T, sim)
